# baseline (speedup 1.0000x reference)
import functools

import jax
import jax.numpy as jnp
from jax import lax
from jax.experimental import pallas as pl
from jax.experimental.pallas import tpu as pltpu
from jax.experimental.pallas import tpu_sc as plsc

ROWS = 64
VOCAB = 100000
NUM_CORES = 2
NUM_SUBCORES = 16
NUM_GROUPS = 8
GROUP_ROWS = ROWS // NUM_GROUPS
NUM_SLABS = 4
CHUNK_W = 3584
MAIN_COLS = 99968
LAST_OFF = MAIN_COLS - CHUNK_W
CHUNKS_PER_SLAB = 7
CHUNK_VREGS = CHUNK_W // 16
TAIL_W = 128
TAIL_VREGS = TAIL_W // 16
SC_GROUPS = 6
TC_ROW0 = SC_GROUPS * GROUP_ROWS
TC_ROWS = ROWS - TC_ROW0

_mesh = plsc.VectorSubcoreMesh(
    core_axis_name="c", subcore_axis_name="s"
)


@functools.partial(
    pl.kernel,
    out_type=jax.ShapeDtypeStruct((SC_GROUPS, GROUP_ROWS, 128), jnp.int32),
    mesh=_mesh,
    scratch_types=[
        pltpu.VMEM((GROUP_ROWS, CHUNK_W), jnp.float32),
        pltpu.VMEM((GROUP_ROWS, CHUNK_W), jnp.float32),
        pltpu.VMEM((GROUP_ROWS, TAIL_W), jnp.float32),
        pltpu.VMEM((GROUP_ROWS, 128), jnp.float32),
        pltpu.VMEM((GROUP_ROWS, 128), jnp.int32),
        [pltpu.VMEM((GROUP_ROWS, 128), jnp.float32)] * NUM_SLABS,
        [pltpu.VMEM((GROUP_ROWS, 128), jnp.int32)] * NUM_SLABS,
        pltpu.VMEM_SHARED((NUM_SUBCORES, GROUP_ROWS, 128), jnp.float32),
        pltpu.VMEM_SHARED((NUM_SUBCORES, GROUP_ROWS, 128), jnp.int32),
        pltpu.SemaphoreType.DMA,
        pltpu.SemaphoreType.DMA,
        pltpu.SemaphoreType.DMA,
    ],
)
def _argmax_sc(
    x_hbm,
    tail_hbm,
    out_hbm,
    buf0,
    buf1,
    tailbuf,
    resv,
    resi,
    mrgv,
    mrgi,
    shv,
    shi,
    sem0,
    sem1,
    sem2,
):
  core = lax.axis_index("c")
  tile = lax.axis_index("s")
  group = core * (SC_GROUPS // NUM_CORES) + tile // NUM_SLABS
  slab = tile % NUM_SLABS
  active = tile < NUM_SLABS * (SC_GROUPS // NUM_CORES)
  row0 = group * GROUP_ROWS
  lane = lax.iota(jnp.int32, 16)

  def chunk_off(k):
    return jnp.minimum((slab + NUM_SLABS * k) * CHUNK_W, LAST_OFF)

  def chunk_src(k):
    return x_hbm.at[pl.ds(row0, GROUP_ROWS), pl.ds(chunk_off(k), CHUNK_W)]

  bufs = (buf0, buf1)
  sems = (sem0, sem1)

  def _scan_and_publish():
    tail_copy = pltpu.async_copy(
        tail_hbm.at[pl.ds(row0, GROUP_ROWS), :], tailbuf, sem2
    )
    copies = [None, None]
    copies[0] = pltpu.async_copy(chunk_src(0), bufs[0], sems[0])

    accs_v = [
        jnp.full((16,), -jnp.inf, jnp.float32) for _ in range(GROUP_ROWS)
    ]
    accs_i = [jnp.zeros((16,), jnp.int32) for _ in range(GROUP_ROWS)]
    for k in range(CHUNKS_PER_SLAB):
      if k + 1 < CHUNKS_PER_SLAB:
        copies[(k + 1) % 2] = pltpu.async_copy(
            chunk_src(k + 1), bufs[(k + 1) % 2], sems[(k + 1) % 2]
        )
      copies[k % 2].wait()
      base = chunk_off(k) + lane

      @plsc.parallel_loop(
          0, CHUNK_VREGS, unroll=2, carry=tuple(accs_v) + tuple(accs_i)
      )
      def carry(i, c, buf=bufs[k % 2], base=base):
        vs = list(c[:GROUP_ROWS])
        idxs = list(c[GROUP_ROWS:])
        idx = base + i * 16
        for r in range(GROUP_ROWS):
          v = buf[r, pl.ds(i * 16, 16)]
          m = v > vs[r]
          vs[r] = jnp.where(m, v, vs[r])
          idxs[r] = jnp.where(m, idx, idxs[r])
        return tuple(vs) + tuple(idxs)

      accs_v = list(carry[:GROUP_ROWS])
      accs_i = list(carry[GROUP_ROWS:])
    acc = tuple(accs_v) + tuple(accs_i)

    tail_copy.wait()

    def tail_body(i, c):
      vs = list(c[:GROUP_ROWS])
      idxs = list(c[GROUP_ROWS:])
      idx = lane + MAIN_COLS + i * 16
      for r in range(GROUP_ROWS):
        v = tailbuf[r, pl.ds(i * 16, 16)]
        m = v > vs[r]
        vs[r] = jnp.where(m, v, vs[r])
        idxs[r] = jnp.where(m, idx, idxs[r])
      return tuple(vs) + tuple(idxs)

    acc = lax.fori_loop(0, TAIL_VREGS, tail_body, acc)

    for r in range(GROUP_ROWS):
      resv[r, pl.ds(0, 16)] = acc[r]
      resi[r, pl.ds(0, 16)] = acc[GROUP_ROWS + r]
    pltpu.sync_copy(resv, shv.at[tile])
    pltpu.sync_copy(resi, shi.at[tile])

  pl.when(active)(_scan_and_publish)
  plsc.subcore_barrier()

  @pl.when(active & (slab == 0))
  def _merge():
    for s in range(NUM_SLABS):
      pltpu.sync_copy(shv.at[tile + s], mrgv[s])
      pltpu.sync_copy(shi.at[tile + s], mrgi[s])
    for r in range(GROUP_ROWS):
      cv = mrgv[0][r, pl.ds(0, 16)]
      ci = mrgi[0][r, pl.ds(0, 16)]
      for s in range(1, NUM_SLABS):
        v = mrgv[s][r, pl.ds(0, 16)]
        i = mrgi[s][r, pl.ds(0, 16)]
        take = (v > cv) | ((v == cv) & (i < ci))
        cv = jnp.where(take, v, cv)
        ci = jnp.where(take, i, ci)
      for sh in (8, 4, 2, 1):
        perm = lane ^ sh
        ov = cv.at[perm].get(mode="promise_in_bounds")
        oi = ci.at[perm].get(mode="promise_in_bounds")
        take = (ov > cv) | ((ov == cv) & (oi < ci))
        cv = jnp.where(take, ov, cv)
        ci = jnp.where(take, oi, ci)
      resi[r, pl.ds(0, 16)] = ci
    pltpu.sync_copy(resi, out_hbm.at[group])


def _tc_argmax_body(x_ref, o_ref):
  x = x_ref[...]
  iota = lax.broadcasted_iota(jnp.int32, (TC_ROWS, VOCAB), 1)
  m = jnp.max(x, axis=1, keepdims=True)
  cand = jnp.where(x == m, iota, jnp.int32(2**31 - 1))
  o_ref[...] = jnp.min(cand, axis=1, keepdims=True)


_tc_argmax = pl.pallas_call(
    _tc_argmax_body,
    out_shape=jax.ShapeDtypeStruct((TC_ROWS, 1), jnp.int32),
)


def kernel(m_logits):
  tail = jnp.pad(
      m_logits[:, MAIN_COLS:],
      ((0, 0), (0, TAIL_W - (VOCAB - MAIN_COLS))),
      constant_values=-jnp.inf,
  )
  sc_out = _argmax_sc(m_logits, tail)
  tc_out = _tc_argmax(m_logits[TC_ROW0:])
  return jnp.concatenate(
      [sc_out[:, :, 0].reshape(TC_ROW0, 1), tc_out], axis=0
  )


# speedup vs baseline: 1.1176x; 1.1176x over previous
import functools

import jax
import jax.numpy as jnp
from jax import lax
from jax.experimental import pallas as pl
from jax.experimental.pallas import tpu as pltpu
from jax.experimental.pallas import tpu_sc as plsc

ROWS = 64
VOCAB = 100000
NUM_CORES = 2
NUM_SUBCORES = 16
NUM_GROUPS = 8
GROUP_ROWS = ROWS // NUM_GROUPS
NUM_SLABS = 8
CHUNK_W = 3200
MAIN_COLS = 99968
LAST_OFF = MAIN_COLS - CHUNK_W
CHUNKS_PER_SLAB = 4
CHUNK_VREGS = CHUNK_W // 16
TAIL_W = 128
TAIL_VREGS = TAIL_W // 16
SC_GROUPS = 4
TC_ROW0 = SC_GROUPS * GROUP_ROWS
TC_ROWS = ROWS - TC_ROW0
TC_BLOCK_ROWS = 8

_mesh = plsc.VectorSubcoreMesh(
    core_axis_name="c", subcore_axis_name="s"
)


@functools.partial(
    pl.kernel,
    out_type=jax.ShapeDtypeStruct((SC_GROUPS, GROUP_ROWS, 128), jnp.int32),
    mesh=_mesh,
    scratch_types=[
        pltpu.VMEM((GROUP_ROWS, CHUNK_W), jnp.float32),
        pltpu.VMEM((GROUP_ROWS, CHUNK_W), jnp.float32),
        pltpu.VMEM((GROUP_ROWS, TAIL_W), jnp.float32),
        pltpu.VMEM((GROUP_ROWS, 128), jnp.float32),
        pltpu.VMEM((GROUP_ROWS, 128), jnp.int32),
        [pltpu.VMEM((GROUP_ROWS, 128), jnp.float32)] * NUM_SLABS,
        [pltpu.VMEM((GROUP_ROWS, 128), jnp.int32)] * NUM_SLABS,
        pltpu.VMEM_SHARED((NUM_SUBCORES, GROUP_ROWS, 128), jnp.float32),
        pltpu.VMEM_SHARED((NUM_SUBCORES, GROUP_ROWS, 128), jnp.int32),
        pltpu.SemaphoreType.DMA,
        pltpu.SemaphoreType.DMA,
        pltpu.SemaphoreType.DMA,
    ],
)
def _argmax_sc(
    x_hbm,
    tail_hbm,
    out_hbm,
    buf0,
    buf1,
    tailbuf,
    resv,
    resi,
    mrgv,
    mrgi,
    shv,
    shi,
    sem0,
    sem1,
    sem2,
):
  core = lax.axis_index("c")
  tile = lax.axis_index("s")
  group = core * (SC_GROUPS // NUM_CORES) + tile // NUM_SLABS
  slab = tile % NUM_SLABS
  active = tile < NUM_SLABS * (SC_GROUPS // NUM_CORES)
  row0 = group * GROUP_ROWS
  lane = lax.iota(jnp.int32, 16)

  def chunk_off(k):
    return jnp.minimum((slab + NUM_SLABS * k) * CHUNK_W, LAST_OFF)

  def chunk_src(k):
    return x_hbm.at[pl.ds(row0, GROUP_ROWS), pl.ds(chunk_off(k), CHUNK_W)]

  bufs = (buf0, buf1)
  sems = (sem0, sem1)

  def _scan_and_publish():
    tail_copy = pltpu.async_copy(
        tail_hbm.at[pl.ds(row0, GROUP_ROWS), :], tailbuf, sem2
    )
    copies = [None, None]
    copies[0] = pltpu.async_copy(chunk_src(0), bufs[0], sems[0])

    accs_v = [
        jnp.full((16,), -jnp.inf, jnp.float32) for _ in range(GROUP_ROWS)
    ]
    accs_i = [jnp.zeros((16,), jnp.int32) for _ in range(GROUP_ROWS)]
    for k in range(CHUNKS_PER_SLAB):
      if k + 1 < CHUNKS_PER_SLAB:
        copies[(k + 1) % 2] = pltpu.async_copy(
            chunk_src(k + 1), bufs[(k + 1) % 2], sems[(k + 1) % 2]
        )
      copies[k % 2].wait()
      base = chunk_off(k) + lane

      @plsc.parallel_loop(
          0, CHUNK_VREGS, unroll=2, carry=tuple(accs_v) + tuple(accs_i)
      )
      def carry(i, c, buf=bufs[k % 2], base=base):
        vs = list(c[:GROUP_ROWS])
        idxs = list(c[GROUP_ROWS:])
        idx = base + i * 16
        for r in range(GROUP_ROWS):
          v = buf[r, pl.ds(i * 16, 16)]
          m = v > vs[r]
          vs[r] = jnp.where(m, v, vs[r])
          idxs[r] = jnp.where(m, idx, idxs[r])
        return tuple(vs) + tuple(idxs)

      accs_v = list(carry[:GROUP_ROWS])
      accs_i = list(carry[GROUP_ROWS:])
    acc = tuple(accs_v) + tuple(accs_i)

    tail_copy.wait()

    def tail_body(i, c):
      vs = list(c[:GROUP_ROWS])
      idxs = list(c[GROUP_ROWS:])
      idx = lane + MAIN_COLS + i * 16
      for r in range(GROUP_ROWS):
        v = tailbuf[r, pl.ds(i * 16, 16)]
        m = v > vs[r]
        vs[r] = jnp.where(m, v, vs[r])
        idxs[r] = jnp.where(m, idx, idxs[r])
      return tuple(vs) + tuple(idxs)

    acc = lax.fori_loop(0, TAIL_VREGS, tail_body, acc)

    for r in range(GROUP_ROWS):
      resv[r, pl.ds(0, 16)] = acc[r]
      resi[r, pl.ds(0, 16)] = acc[GROUP_ROWS + r]
    pltpu.sync_copy(resv, shv.at[tile])
    pltpu.sync_copy(resi, shi.at[tile])

  pl.when(active)(_scan_and_publish)
  plsc.subcore_barrier()

  @pl.when(active & (slab == 0))
  def _merge():
    for s in range(NUM_SLABS):
      pltpu.sync_copy(shv.at[tile + s], mrgv[s])
      pltpu.sync_copy(shi.at[tile + s], mrgi[s])
    for r in range(GROUP_ROWS):
      cv = mrgv[0][r, pl.ds(0, 16)]
      ci = mrgi[0][r, pl.ds(0, 16)]
      for s in range(1, NUM_SLABS):
        v = mrgv[s][r, pl.ds(0, 16)]
        i = mrgi[s][r, pl.ds(0, 16)]
        take = (v > cv) | ((v == cv) & (i < ci))
        cv = jnp.where(take, v, cv)
        ci = jnp.where(take, i, ci)
      for sh in (8, 4, 2, 1):
        perm = lane ^ sh
        ov = cv.at[perm].get(mode="promise_in_bounds")
        oi = ci.at[perm].get(mode="promise_in_bounds")
        take = (ov > cv) | ((ov == cv) & (oi < ci))
        cv = jnp.where(take, ov, cv)
        ci = jnp.where(take, oi, ci)
      resi[r, pl.ds(0, 16)] = ci
    pltpu.sync_copy(resi, out_hbm.at[group])


def _tc_argmax_body(x_ref, o_ref):
  x = x_ref[...]
  iota = lax.broadcasted_iota(jnp.int32, (TC_BLOCK_ROWS, VOCAB), 1)
  m = jnp.max(x, axis=1, keepdims=True)
  cand = jnp.where(x == m, iota, jnp.int32(2**31 - 1))
  o_ref[...] = jnp.min(cand, axis=1, keepdims=True)


_tc_argmax = pl.pallas_call(
    _tc_argmax_body,
    grid=(TC_ROWS // TC_BLOCK_ROWS,),
    in_specs=[
        pl.BlockSpec(
            (TC_BLOCK_ROWS, VOCAB),
            lambda i: (TC_ROW0 // TC_BLOCK_ROWS + i, 0),
        )
    ],
    out_specs=pl.BlockSpec((TC_BLOCK_ROWS, 1), lambda i: (i, 0)),
    out_shape=jax.ShapeDtypeStruct((TC_ROWS, 1), jnp.int32),
)


def kernel(m_logits):
  tail = jnp.pad(
      m_logits[:, MAIN_COLS:],
      ((0, 0), (0, TAIL_W - (VOCAB - MAIN_COLS))),
      constant_values=-jnp.inf,
  )
  sc_out = _argmax_sc(m_logits, tail)
  tc_out = _tc_argmax(m_logits)
  return jnp.concatenate(
      [sc_out[:, :, 0].reshape(TC_ROW0, 1), tc_out], axis=0
  )
